# baseline (speedup 1.0000x reference)
import functools

import jax
import jax.numpy as jnp
from jax import lax
from jax.experimental import pallas as pl
from jax.experimental.pallas import tpu as pltpu
from jax.experimental.pallas import tpu_sc as plsc

_VOCAB = 100000
_D = 64
_H = 64
_B = 1024
_LC = 200
_LQ = 20
_P = 10
_N = _B * (_LC + _LQ)

_NC = 2
_NS = 16
_NW = _NC * _NS
_ROWS_PER_WORKER = _N // _NW
_CHUNK = 88
_NCHUNKS = _ROWS_PER_WORKER // _CHUNK
_NBUF = 4


def _fixup_body(qw_ref, ph_ref, rw_ref, out_ref):
    qw = qw_ref[...]
    ph = ph_ref[...]
    repl = jnp.zeros_like(qw)
    for p in range(_P):
        col = rw_ref[:, p : p + 1]
        repl = jnp.where(ph == p, col, repl)
    out_ref[...] = jnp.where(repl > 0, repl, qw)


def _fixup(qw_idxs, qw_to_phrases, rw_idxs):
    return pl.pallas_call(
        _fixup_body,
        out_shape=jax.ShapeDtypeStruct((_B, _LQ), jnp.int32),
    )(qw_idxs, qw_to_phrases, rw_idxs)


def _transform_body(wv_ref, pw_ref, gw_ref, gb_ref, tw_ref, tb_ref, out_ref):
    e = jnp.dot(
        wv_ref[...], pw_ref[...], preferred_element_type=jnp.float32)
    for i in range(2):
        g = jax.nn.sigmoid(
            jnp.dot(e, gw_ref[i], preferred_element_type=jnp.float32)
            + gb_ref[i : i + 1, :]
        )
        t = jax.nn.relu(
            jnp.dot(e, tw_ref[i], preferred_element_type=jnp.float32)
            + tb_ref[i : i + 1, :]
        )
        e = g * t + (1.0 - g) * e
    out_ref[...] = jnp.concatenate([e, jnp.zeros_like(e)], axis=1)


_TROWS = 2000


def _transform_table(word_vectors, proj_w, gw, gb, tw, tb):
    grid = _VOCAB // _TROWS
    full = lambda *shape: pl.BlockSpec(shape, lambda i: (0,) * len(shape))
    return pl.pallas_call(
        _transform_body,
        grid=(grid,),
        in_specs=[
            pl.BlockSpec((_TROWS, _D), lambda i: (i, 0)),
            full(_D, _H),
            full(2, _H, _H),
            full(2, _H),
            full(2, _H, _H),
            full(2, _H),
        ],
        out_specs=pl.BlockSpec((_TROWS, 2 * _H), lambda i: (i, 0)),
        out_shape=jax.ShapeDtypeStruct((_VOCAB, 2 * _H), jnp.float32),
    )(word_vectors, proj_w, gw, gb, tw, tb)


def _sc_gather(table, idx):
    mesh = plsc.VectorSubcoreMesh(core_axis_name="c", subcore_axis_name="s")

    @functools.partial(
        pl.kernel,
        mesh=mesh,
        out_type=jax.ShapeDtypeStruct((_N, 2 * _H), jnp.float32),
        scratch_types=(
            [pltpu.VMEM((_CHUNK,), jnp.int32) for _ in range(_NBUF)]
            + [pltpu.VMEM((_CHUNK, 2 * _H), jnp.float32) for _ in range(_NBUF)]
            + [pltpu.SemaphoreType.DMA for _ in range(2 * _NBUF)]
        ),
    )
    def k(table_hbm, idx_hbm, out_hbm, *scratch):
        idx_v = scratch[:_NBUF]
        rows_v = scratch[_NBUF : 2 * _NBUF]
        gsem = scratch[2 * _NBUF : 3 * _NBUF]
        osem = scratch[3 * _NBUF : 4 * _NBUF]
        wid = lax.axis_index("s") * _NC + lax.axis_index("c")
        base = wid * _ROWS_PER_WORKER

        def start_gather(ci, b):
            off = base + ci * _CHUNK
            pltpu.sync_copy(idx_hbm.at[pl.ds(off, _CHUNK)], idx_v[b])
            pltpu.async_copy(table_hbm.at[idx_v[b]], rows_v[b], gsem[b])

        def wait_gather(b):
            pltpu.make_async_copy(table_hbm.at[idx_v[b]], rows_v[b],
                                  gsem[b]).wait()

        def start_out(ci, b):
            off = base + ci * _CHUNK
            pltpu.async_copy(rows_v[b], out_hbm.at[pl.ds(off, _CHUNK)],
                             osem[b])

        def wait_out(ci, b):
            off = base + ci * _CHUNK
            pltpu.make_async_copy(rows_v[b], out_hbm.at[pl.ds(off, _CHUNK)],
                                  osem[b]).wait()

        for b in range(_NBUF):
            start_gather(b, b)

        @pl.loop(0, _NCHUNKS - _NBUF, step=_NBUF)
        def _(k):
            for b in range(_NBUF):
                wait_gather(b)
                start_out(k + b, b)
            for b in range(_NBUF):
                wait_out(k + b, b)
                start_gather(k + b + _NBUF, b)

        for b in range(_NBUF):
            ci = _NCHUNKS - _NBUF + b
            wait_gather(b)
            start_out(ci, b)
        for b in range(_NBUF):
            wait_out(_NCHUNKS - _NBUF + b, b)

    return k(table, idx)


_GB = 8


def _format_body(in_ref, out_ref):
    for g in range(_GB):
        out_ref[g] = in_ref[pl.ds(g * (_LC + _LQ), _LC + _LQ), : _H]


def _format_output(flat):
    lt = _LC + _LQ
    return pl.pallas_call(
        _format_body,
        grid=(_B // _GB,),
        in_specs=[pl.BlockSpec((_GB * lt, 2 * _H), lambda i: (i, 0))],
        out_specs=pl.BlockSpec((_GB, lt, _H), lambda i: (i, 0, 0)),
        out_shape=jax.ShapeDtypeStruct((_B, lt, _H), jnp.float32),
    )(flat)


def kernel(cw_idxs, qw_idxs, qw_to_phrases, rw_idxs, word_vectors, proj_w,
           hwy_gate_w, hwy_gate_b, hwy_trans_w, hwy_trans_b):
    cw = cw_idxs.astype(jnp.int32)
    qw = qw_idxs.astype(jnp.int32)
    ph = qw_to_phrases.astype(jnp.int32)
    rw = rw_idxs.astype(jnp.int32)

    new_qw = _fixup(qw, ph, rw)
    table2 = _transform_table(word_vectors, proj_w, hwy_gate_w, hwy_gate_b,
                              hwy_trans_w, hwy_trans_b)
    idx = jnp.concatenate([cw, new_qw], axis=1).reshape(-1)
    out = _sc_gather(table2, idx)
    return _format_output(out)


# speedup vs baseline: 1.5406x; 1.5406x over previous
import functools

import jax
import jax.numpy as jnp
from jax import lax
from jax.experimental import pallas as pl
from jax.experimental.pallas import tpu as pltpu
from jax.experimental.pallas import tpu_sc as plsc

_VOCAB = 100000
_D = 64
_H = 64
_B = 1024
_LC = 200
_LQ = 20
_P = 10
_N = _B * (_LC + _LQ)

_NC = 2
_NS = 16
_NW = _NC * _NS
_ROWS_PER_WORKER = _N // _NW
_CHUNK = 88
_NCHUNKS = _ROWS_PER_WORKER // _CHUNK
_NBUF = 4


def _fixup_body(qw_ref, ph_ref, rw_ref, out_ref):
    qw = qw_ref[...]
    ph = ph_ref[...]
    repl = jnp.zeros_like(qw)
    for p in range(_P):
        col = rw_ref[:, p : p + 1]
        repl = jnp.where(ph == p, col, repl)
    out_ref[...] = jnp.where(repl > 0, repl, qw)


def _fixup(qw_idxs, qw_to_phrases, rw_idxs):
    return pl.pallas_call(
        _fixup_body,
        out_shape=jax.ShapeDtypeStruct((_B, _LQ), jnp.int32),
    )(qw_idxs, qw_to_phrases, rw_idxs)


def _transform_body(wv_ref, pw_ref, gw_ref, gb_ref, tw_ref, tb_ref, out_ref):
    e = jnp.dot(
        wv_ref[...], pw_ref[...], preferred_element_type=jnp.float32)
    for i in range(2):
        g = jax.nn.sigmoid(
            jnp.dot(e, gw_ref[i], preferred_element_type=jnp.float32)
            + gb_ref[i : i + 1, :]
        )
        t = jax.nn.relu(
            jnp.dot(e, tw_ref[i], preferred_element_type=jnp.float32)
            + tb_ref[i : i + 1, :]
        )
        e = g * t + (1.0 - g) * e
    out_ref[...] = jnp.concatenate([e, jnp.zeros_like(e)], axis=1)


_TROWS = 2000


def _transform_table(word_vectors, proj_w, gw, gb, tw, tb):
    grid = _VOCAB // _TROWS
    full = lambda *shape: pl.BlockSpec(shape, lambda i: (0,) * len(shape))
    return pl.pallas_call(
        _transform_body,
        grid=(grid,),
        in_specs=[
            pl.BlockSpec((_TROWS, _D), lambda i: (i, 0)),
            full(_D, _H),
            full(2, _H, _H),
            full(2, _H),
            full(2, _H, _H),
            full(2, _H),
        ],
        out_specs=pl.BlockSpec((_TROWS, 2 * _H), lambda i: (i, 0)),
        out_shape=jax.ShapeDtypeStruct((_VOCAB, 2 * _H), jnp.float32),
    )(word_vectors, proj_w, gw, gb, tw, tb)


_LT = _LC + _LQ
_LPAD = 224
_GROUPS_PER_WORKER = _B // (2 * _NW)
_CHUNKS4 = (
    (0, 128, 0, 0, 128),
    (128, 96, 0, 128, 92),
    (_LPAD, 128, 1, 0, 128),
    (_LPAD + 128, 96, 1, 128, 92),
)


def _sc_gather(table, idx):
    mesh = plsc.VectorSubcoreMesh(core_axis_name="c", subcore_axis_name="s")

    @functools.partial(
        pl.kernel,
        mesh=mesh,
        out_type=jax.ShapeDtypeStruct((_B, _LT, 2 * _H), jnp.float32),
        scratch_types=(
            [pltpu.VMEM((c[1],), jnp.int32) for c in _CHUNKS4]
            + [pltpu.VMEM((c[1], 2 * _H), jnp.float32) for c in _CHUNKS4]
            + [pltpu.SemaphoreType.DMA for _ in range(8)]
        ),
    )
    def k(table_hbm, idx_hbm, out_hbm, *scratch):
        idx_v = scratch[:4]
        rows_v = scratch[4:8]
        gsem = scratch[8:12]
        osem = scratch[12:16]
        wid = lax.axis_index("s") * _NC + lax.axis_index("c")
        idx_base = wid * (2 * _LPAD * _GROUPS_PER_WORKER)
        batch_base = wid * (2 * _GROUPS_PER_WORKER)

        def start_gather(g, b):
            off = idx_base + g * (2 * _LPAD) + _CHUNKS4[b][0]
            pltpu.sync_copy(idx_hbm.at[pl.ds(off, _CHUNKS4[b][1])], idx_v[b])
            pltpu.async_copy(table_hbm.at[idx_v[b]], rows_v[b], gsem[b])

        def wait_gather(b):
            pltpu.make_async_copy(table_hbm.at[idx_v[b]], rows_v[b],
                                  gsem[b]).wait()

        def start_out(g, b):
            _, _, db, r0, nr = _CHUNKS4[b]
            dst = out_hbm.at[batch_base + 2 * g + db, pl.ds(r0, nr), :]
            pltpu.async_copy(rows_v[b].at[pl.ds(0, nr)], dst, osem[b])

        def wait_out(g, b):
            _, _, db, r0, nr = _CHUNKS4[b]
            dst = out_hbm.at[batch_base + 2 * g + db, pl.ds(r0, nr), :]
            pltpu.make_async_copy(rows_v[b].at[pl.ds(0, nr)], dst,
                                  osem[b]).wait()

        for b in range(4):
            start_gather(0, b)

        @pl.loop(0, _GROUPS_PER_WORKER - 1)
        def _(g):
            for b in range(4):
                wait_gather(b)
                start_out(g, b)
            for b in range(4):
                wait_out(g, b)
                start_gather(g + 1, b)

        gl = _GROUPS_PER_WORKER - 1
        for b in range(4):
            wait_gather(b)
            start_out(gl, b)
        for b in range(4):
            wait_out(gl, b)

    return k(table, idx)


def kernel(cw_idxs, qw_idxs, qw_to_phrases, rw_idxs, word_vectors, proj_w,
           hwy_gate_w, hwy_gate_b, hwy_trans_w, hwy_trans_b):
    cw = cw_idxs.astype(jnp.int32)
    qw = qw_idxs.astype(jnp.int32)
    ph = qw_to_phrases.astype(jnp.int32)
    rw = rw_idxs.astype(jnp.int32)

    new_qw = _fixup(qw, ph, rw)
    table2 = _transform_table(word_vectors, proj_w, hwy_gate_w, hwy_gate_b,
                              hwy_trans_w, hwy_trans_b)
    idx = jnp.concatenate([cw, new_qw, cw[:, :_LPAD - _LT]],
                          axis=1).reshape(-1)
    out = _sc_gather(table2, idx)
    return out[:, :, :_H]


# speedup vs baseline: 1.6206x; 1.0519x over previous
import functools

import jax
import jax.numpy as jnp
from jax import lax
from jax.experimental import pallas as pl
from jax.experimental.pallas import tpu as pltpu
from jax.experimental.pallas import tpu_sc as plsc

_VOCAB = 100000
_D = 64
_H = 64
_B = 1024
_LC = 200
_LQ = 20
_P = 10
_N = _B * (_LC + _LQ)

_NC = 2
_NS = 16
_NW = _NC * _NS
_ROWS_PER_WORKER = _N // _NW
_CHUNK = 88
_NCHUNKS = _ROWS_PER_WORKER // _CHUNK
_NBUF = 4


def _fixup_body(qw_ref, ph_ref, rw_ref, out_ref):
    qw = qw_ref[...]
    ph = ph_ref[...]
    repl = jnp.zeros_like(qw)
    for p in range(_P):
        col = rw_ref[:, p : p + 1]
        repl = jnp.where(ph == p, col, repl)
    out_ref[...] = jnp.where(repl > 0, repl, qw)


def _fixup(qw_idxs, qw_to_phrases, rw_idxs):
    return pl.pallas_call(
        _fixup_body,
        out_shape=jax.ShapeDtypeStruct((_B, _LQ), jnp.int32),
    )(qw_idxs, qw_to_phrases, rw_idxs)


def _transform_body(wv_ref, pw_ref, gtw_ref, gtb_ref, out_ref):
    e = jnp.dot(
        wv_ref[...], pw_ref[...], preferred_element_type=jnp.float32)
    for i in range(2):
        gt = jnp.dot(e, gtw_ref[i], preferred_element_type=jnp.float32)
        gt = gt + gtb_ref[i : i + 1, :]
        g = jax.nn.sigmoid(gt[:, :_H])
        t = jax.nn.relu(gt[:, _H:])
        e = g * t + (1.0 - g) * e
    out_ref[...] = jnp.concatenate([e, jnp.zeros_like(e)], axis=1)


_TROWS = 4000


def _transform_table(word_vectors, proj_w, gw, gb, tw, tb):
    gtw = jnp.concatenate([gw, tw], axis=2)
    gtb = jnp.concatenate([gb, tb], axis=1)
    grid = _VOCAB // _TROWS
    full = lambda *shape: pl.BlockSpec(shape, lambda i: (0,) * len(shape))
    return pl.pallas_call(
        _transform_body,
        grid=(grid,),
        in_specs=[
            pl.BlockSpec((_TROWS, _D), lambda i: (i, 0)),
            full(_D, _H),
            full(2, _H, 2 * _H),
            full(2, 2 * _H),
        ],
        out_specs=pl.BlockSpec((_TROWS, 2 * _H), lambda i: (i, 0)),
        out_shape=jax.ShapeDtypeStruct((_VOCAB, 2 * _H), jnp.float32),
    )(word_vectors, proj_w, gtw, gtb)


_LT = _LC + _LQ
_LPAD = 224
_GROUPS_PER_WORKER = _B // (2 * _NW)
_CHUNKS4 = (
    (0, 128, 0, 0, 128),
    (128, 96, 0, 128, 92),
    (_LPAD, 128, 1, 0, 128),
    (_LPAD + 128, 96, 1, 128, 92),
)


def _sc_gather(table, idx):
    mesh = plsc.VectorSubcoreMesh(core_axis_name="c", subcore_axis_name="s")

    @functools.partial(
        pl.kernel,
        mesh=mesh,
        out_type=jax.ShapeDtypeStruct((_B, _LT, 2 * _H), jnp.float32),
        scratch_types=(
            [pltpu.VMEM((c[1],), jnp.int32) for c in _CHUNKS4]
            + [pltpu.VMEM((c[1], 2 * _H), jnp.float32) for c in _CHUNKS4]
            + [pltpu.SemaphoreType.DMA for _ in range(8)]
        ),
    )
    def k(table_hbm, idx_hbm, out_hbm, *scratch):
        idx_v = scratch[:4]
        rows_v = scratch[4:8]
        gsem = scratch[8:12]
        osem = scratch[12:16]
        wid = lax.axis_index("s") * _NC + lax.axis_index("c")
        idx_base = wid * (2 * _LPAD * _GROUPS_PER_WORKER)
        batch_base = wid * (2 * _GROUPS_PER_WORKER)

        def start_gather(g, b):
            off = idx_base + g * (2 * _LPAD) + _CHUNKS4[b][0]
            pltpu.sync_copy(idx_hbm.at[pl.ds(off, _CHUNKS4[b][1])], idx_v[b])
            pltpu.async_copy(table_hbm.at[idx_v[b]], rows_v[b], gsem[b])

        def wait_gather(b):
            pltpu.make_async_copy(table_hbm.at[idx_v[b]], rows_v[b],
                                  gsem[b]).wait()

        def start_out(g, b):
            _, _, db, r0, nr = _CHUNKS4[b]
            dst = out_hbm.at[batch_base + 2 * g + db, pl.ds(r0, nr), :]
            pltpu.async_copy(rows_v[b].at[pl.ds(0, nr)], dst, osem[b])

        def wait_out(g, b):
            _, _, db, r0, nr = _CHUNKS4[b]
            dst = out_hbm.at[batch_base + 2 * g + db, pl.ds(r0, nr), :]
            pltpu.make_async_copy(rows_v[b].at[pl.ds(0, nr)], dst,
                                  osem[b]).wait()

        for b in range(4):
            start_gather(0, b)

        @pl.loop(0, _GROUPS_PER_WORKER - 1)
        def _(g):
            for b in range(4):
                wait_gather(b)
                start_out(g, b)
            for b in range(4):
                wait_out(g, b)
                start_gather(g + 1, b)

        gl = _GROUPS_PER_WORKER - 1
        for b in range(4):
            wait_gather(b)
            start_out(gl, b)
        for b in range(4):
            wait_out(gl, b)

    return k(table, idx)


def kernel(cw_idxs, qw_idxs, qw_to_phrases, rw_idxs, word_vectors, proj_w,
           hwy_gate_w, hwy_gate_b, hwy_trans_w, hwy_trans_b):
    cw = cw_idxs.astype(jnp.int32)
    qw = qw_idxs.astype(jnp.int32)
    ph = qw_to_phrases.astype(jnp.int32)
    rw = rw_idxs.astype(jnp.int32)

    new_qw = _fixup(qw, ph, rw)
    table2 = _transform_table(word_vectors, proj_w, hwy_gate_w, hwy_gate_b,
                              hwy_trans_w, hwy_trans_b)
    idx = jnp.concatenate([cw, new_qw, cw[:, :_LPAD - _LT]],
                          axis=1).reshape(-1)
    out = _sc_gather(table2, idx)
    return out[:, :, :_H]


# speedup vs baseline: 1.8375x; 1.1338x over previous
import functools

import jax
import jax.numpy as jnp
from jax import lax
from jax.experimental import pallas as pl
from jax.experimental.pallas import tpu as pltpu
from jax.experimental.pallas import tpu_sc as plsc

_VOCAB = 100000
_D = 64
_H = 64
_B = 1024
_LC = 200
_LQ = 20
_P = 10
_N = _B * (_LC + _LQ)

_NC = 2
_NS = 16
_NW = _NC * _NS
_ROWS_PER_WORKER = _N // _NW
_CHUNK = 88
_NCHUNKS = _ROWS_PER_WORKER // _CHUNK
_NBUF = 4


def _fixup_body(qw_ref, ph_ref, rw_ref, out_ref):
    qw = qw_ref[...]
    ph = ph_ref[...]
    repl = jnp.zeros_like(qw)
    for p in range(_P):
        col = rw_ref[:, p : p + 1]
        repl = jnp.where(ph == p, col, repl)
    out_ref[...] = jnp.where(repl > 0, repl, qw)


def _fixup(qw_idxs, qw_to_phrases, rw_idxs):
    return pl.pallas_call(
        _fixup_body,
        out_shape=jax.ShapeDtypeStruct((_B, _LQ), jnp.int32),
    )(qw_idxs, qw_to_phrases, rw_idxs)


def _transform_body(wv_ref, pw_ref, gtw_ref, gtb_ref, out_ref):
    e = lax.dot_general(
        wv_ref[...], pw_ref[...],
        dimension_numbers=(((0,), (0,)), ((), ())),
        preferred_element_type=jnp.float32)
    for i in range(2):
        gt = jnp.dot(e, gtw_ref[i], preferred_element_type=jnp.float32)
        gt = gt + gtb_ref[i : i + 1, :]
        g = jax.nn.sigmoid(gt[:, :_H])
        t = jax.nn.relu(gt[:, _H:])
        e = g * t + (1.0 - g) * e
    out_ref[...] = jnp.concatenate([e, jnp.zeros_like(e)], axis=1)


_TROWS = 4096


def _transform_table(word_vectors, proj_w, gw, gb, tw, tb):
    gtw = jnp.concatenate([gw, tw], axis=2)
    gtb = jnp.concatenate([gb, tb], axis=1)
    grid = -(-_VOCAB // _TROWS)
    full = lambda *shape: pl.BlockSpec(shape, lambda i: (0,) * len(shape))
    return pl.pallas_call(
        _transform_body,
        grid=(grid,),
        in_specs=[
            pl.BlockSpec((_D, _TROWS), lambda i: (0, i)),
            full(_D, _H),
            full(2, _H, 2 * _H),
            full(2, 2 * _H),
        ],
        out_specs=pl.BlockSpec((_TROWS, 2 * _H), lambda i: (i, 0)),
        out_shape=jax.ShapeDtypeStruct((_VOCAB, 2 * _H), jnp.float32),
    )(word_vectors.T, proj_w, gtw, gtb)


_LT = _LC + _LQ
_LPAD = 224
_GROUPS_PER_WORKER = _B // (2 * _NW)
_CHUNKS4 = (
    (0, 128, 0, 0, 128),
    (128, 96, 0, 128, 92),
    (_LPAD, 128, 1, 0, 128),
    (_LPAD + 128, 96, 1, 128, 92),
)


def _sc_gather(table, idx):
    mesh = plsc.VectorSubcoreMesh(core_axis_name="c", subcore_axis_name="s")

    @functools.partial(
        pl.kernel,
        mesh=mesh,
        out_type=jax.ShapeDtypeStruct((_B, _LT, 2 * _H), jnp.float32),
        scratch_types=(
            [pltpu.VMEM((c[1],), jnp.int32) for c in _CHUNKS4]
            + [pltpu.VMEM((c[1], 2 * _H), jnp.float32) for c in _CHUNKS4]
            + [pltpu.SemaphoreType.DMA for _ in range(8)]
        ),
    )
    def k(table_hbm, idx_hbm, out_hbm, *scratch):
        idx_v = scratch[:4]
        rows_v = scratch[4:8]
        gsem = scratch[8:12]
        osem = scratch[12:16]
        wid = lax.axis_index("s") * _NC + lax.axis_index("c")
        idx_base = wid * (2 * _LPAD * _GROUPS_PER_WORKER)
        batch_base = wid * (2 * _GROUPS_PER_WORKER)

        def start_gather(g, b):
            off = idx_base + g * (2 * _LPAD) + _CHUNKS4[b][0]
            pltpu.sync_copy(idx_hbm.at[pl.ds(off, _CHUNKS4[b][1])], idx_v[b])
            pltpu.async_copy(table_hbm.at[idx_v[b]], rows_v[b], gsem[b])

        def wait_gather(b):
            pltpu.make_async_copy(table_hbm.at[idx_v[b]], rows_v[b],
                                  gsem[b]).wait()

        def start_out(g, b):
            _, _, db, r0, nr = _CHUNKS4[b]
            dst = out_hbm.at[batch_base + 2 * g + db, pl.ds(r0, nr), :]
            pltpu.async_copy(rows_v[b].at[pl.ds(0, nr)], dst, osem[b])

        def wait_out(g, b):
            _, _, db, r0, nr = _CHUNKS4[b]
            dst = out_hbm.at[batch_base + 2 * g + db, pl.ds(r0, nr), :]
            pltpu.make_async_copy(rows_v[b].at[pl.ds(0, nr)], dst,
                                  osem[b]).wait()

        for b in range(4):
            start_gather(0, b)

        @pl.loop(0, _GROUPS_PER_WORKER - 1)
        def _(g):
            for b in range(4):
                wait_gather(b)
                start_out(g, b)
            for b in range(4):
                wait_out(g, b)
                start_gather(g + 1, b)

        gl = _GROUPS_PER_WORKER - 1
        for b in range(4):
            wait_gather(b)
            start_out(gl, b)
        for b in range(4):
            wait_out(gl, b)

    return k(table, idx)


def kernel(cw_idxs, qw_idxs, qw_to_phrases, rw_idxs, word_vectors, proj_w,
           hwy_gate_w, hwy_gate_b, hwy_trans_w, hwy_trans_b):
    cw = cw_idxs.astype(jnp.int32)
    qw = qw_idxs.astype(jnp.int32)
    ph = qw_to_phrases.astype(jnp.int32)
    rw = rw_idxs.astype(jnp.int32)

    new_qw = _fixup(qw, ph, rw)
    table2 = _transform_table(word_vectors, proj_w, hwy_gate_w, hwy_gate_b,
                              hwy_trans_w, hwy_trans_b)
    idx = jnp.concatenate([cw, new_qw, cw[:, :_LPAD - _LT]],
                          axis=1).reshape(-1)
    out = _sc_gather(table2, idx)
    return out[:, :, :_H]


# speedup vs baseline: 1.8507x; 1.0072x over previous
import functools

import jax
import jax.numpy as jnp
from jax import lax
from jax.experimental import pallas as pl
from jax.experimental.pallas import tpu as pltpu
from jax.experimental.pallas import tpu_sc as plsc

_VOCAB = 100000
_D = 64
_H = 64
_B = 1024
_LC = 200
_LQ = 20
_P = 10
_N = _B * (_LC + _LQ)

_NC = 2
_NS = 16
_NW = _NC * _NS
_ROWS_PER_WORKER = _N // _NW
_CHUNK = 88
_NCHUNKS = _ROWS_PER_WORKER // _CHUNK
_NBUF = 4


def _fixup_body(qw_ref, ph_ref, rw_ref, out_ref):
    qw = qw_ref[...]
    ph = ph_ref[...]
    repl = jnp.zeros_like(qw)
    for p in range(_P):
        col = rw_ref[:, p : p + 1]
        repl = jnp.where(ph == p, col, repl)
    out_ref[...] = jnp.where(repl > 0, repl, qw)


def _fixup(qw_idxs, qw_to_phrases, rw_idxs):
    return pl.pallas_call(
        _fixup_body,
        out_shape=jax.ShapeDtypeStruct((_B, _LQ), jnp.int32),
    )(qw_idxs, qw_to_phrases, rw_idxs)


def _transform_body(wv_ref, pw_ref, gtw_ref, gtb_ref, out_ref):
    e = lax.dot_general(
        wv_ref[...], pw_ref[...],
        dimension_numbers=(((0,), (0,)), ((), ())),
        preferred_element_type=jnp.float32)
    for i in range(2):
        gt = jnp.dot(e, gtw_ref[i], preferred_element_type=jnp.float32)
        gt = gt + gtb_ref[i : i + 1, :]
        g = jax.nn.sigmoid(gt[:, :_H])
        t = jax.nn.relu(gt[:, _H:])
        e = g * t + (1.0 - g) * e
    out_ref[...] = jnp.concatenate([e, jnp.zeros_like(e)], axis=1)


_TROWS = 4096


def _transform_table(word_vectors, proj_w, gw, gb, tw, tb):
    gtw = jnp.concatenate([gw, tw], axis=2)
    gtb = jnp.concatenate([gb, tb], axis=1)
    grid = -(-_VOCAB // _TROWS)
    full = lambda *shape: pl.BlockSpec(shape, lambda i: (0,) * len(shape))
    return pl.pallas_call(
        _transform_body,
        grid=(grid,),
        in_specs=[
            pl.BlockSpec((_D, _TROWS), lambda i: (0, i)),
            full(_D, _H),
            full(2, _H, 2 * _H),
            full(2, 2 * _H),
        ],
        out_specs=pl.BlockSpec((_TROWS, 2 * _H), lambda i: (i, 0)),
        out_shape=jax.ShapeDtypeStruct((_VOCAB, 2 * _H), jnp.float32),
    )(word_vectors.T, proj_w, gtw, gtb)


_LT = _LC + _LQ
_LPAD = 224
_GROUPS_PER_WORKER = _B // (2 * _NW)
_CHUNKS4 = (
    (0, 128, 0, 0, 128),
    (128, 96, 0, 128, 92),
    (_LPAD, 128, 1, 0, 128),
    (_LPAD + 128, 96, 1, 128, 92),
)


def _sc_gather(table, idx):
    mesh = plsc.VectorSubcoreMesh(core_axis_name="c", subcore_axis_name="s")

    @functools.partial(
        pl.kernel,
        mesh=mesh,
        out_type=jax.ShapeDtypeStruct((_B, _LT, 2 * _H), jnp.float32),
        scratch_types=(
            [pltpu.VMEM((_CHUNKS4[b % 4][1],), jnp.int32) for b in range(8)]
            + [pltpu.VMEM((_CHUNKS4[b % 4][1], 2 * _H), jnp.float32)
               for b in range(8)]
            + [pltpu.SemaphoreType.DMA for _ in range(16)]
        ),
    )
    def k(table_hbm, idx_hbm, out_hbm, *scratch):
        idx_v = scratch[:8]
        rows_v = scratch[8:16]
        gsem = scratch[16:24]
        osem = scratch[24:32]
        wid = lax.axis_index("s") * _NC + lax.axis_index("c")
        idx_base = wid * (2 * _LPAD * _GROUPS_PER_WORKER)
        batch_base = wid * (2 * _GROUPS_PER_WORKER)

        def start_gather(g, b):
            pos = b % 4
            off = idx_base + g * (2 * _LPAD) + _CHUNKS4[pos][0]
            pltpu.sync_copy(idx_hbm.at[pl.ds(off, _CHUNKS4[pos][1])], idx_v[b])
            pltpu.async_copy(table_hbm.at[idx_v[b]], rows_v[b], gsem[b])

        def wait_gather(b):
            pltpu.make_async_copy(table_hbm.at[idx_v[b]], rows_v[b],
                                  gsem[b]).wait()

        def start_out(g, b):
            _, _, db, r0, nr = _CHUNKS4[b % 4]
            dst = out_hbm.at[batch_base + 2 * g + db, pl.ds(r0, nr), :]
            pltpu.async_copy(rows_v[b].at[pl.ds(0, nr)], dst, osem[b])

        def wait_out(g, b):
            _, _, db, r0, nr = _CHUNKS4[b % 4]
            dst = out_hbm.at[batch_base + 2 * g + db, pl.ds(r0, nr), :]
            pltpu.make_async_copy(rows_v[b].at[pl.ds(0, nr)], dst,
                                  osem[b]).wait()

        for b in range(8):
            start_gather(b // 4, b)

        @pl.loop(0, _GROUPS_PER_WORKER - 2, step=2)
        def _(k):
            for par in range(2):
                g = k + par
                for pos in range(4):
                    b = par * 4 + pos
                    wait_gather(b)
                    start_out(g, b)
                for pos in range(4):
                    b = par * 4 + pos
                    wait_out(g, b)
                    start_gather(g + 2, b)

        for par in range(2):
            gl = _GROUPS_PER_WORKER - 2 + par
            for pos in range(4):
                b = par * 4 + pos
                wait_gather(b)
                start_out(gl, b)
        for par in range(2):
            gl = _GROUPS_PER_WORKER - 2 + par
            for pos in range(4):
                wait_out(gl, par * 4 + pos)

    return k(table, idx)


def kernel(cw_idxs, qw_idxs, qw_to_phrases, rw_idxs, word_vectors, proj_w,
           hwy_gate_w, hwy_gate_b, hwy_trans_w, hwy_trans_b):
    cw = cw_idxs.astype(jnp.int32)
    qw = qw_idxs.astype(jnp.int32)
    ph = qw_to_phrases.astype(jnp.int32)
    rw = rw_idxs.astype(jnp.int32)

    new_qw = _fixup(qw, ph, rw)
    table2 = _transform_table(word_vectors, proj_w, hwy_gate_w, hwy_gate_b,
                              hwy_trans_w, hwy_trans_b)
    idx = jnp.concatenate([cw, new_qw, cw[:, :_LPAD - _LT]],
                          axis=1).reshape(-1)
    out = _sc_gather(table2, idx)
    return out[:, :, :_H]
